# Initial kernel scaffold; baseline (speedup 1.0000x reference)
import jax
import jax.numpy as jnp
from jax.experimental import pallas as pl


def kernel(x, edge_index, W1, b1, W2, b2):
    raise NotImplementedError("write your pallas kernel here")


# speedup vs baseline: 40.7964x; 40.7964x over previous
import functools

import jax
import jax.numpy as jnp
from jax import lax
from jax.experimental import pallas as pl
from jax.experimental.pallas import tpu as pltpu
from jax.experimental.pallas import tpu_sc as plsc

NC = 2
NS = 16
NW = NC * NS


def _mesh():
    return plsc.VectorSubcoreMesh(
        core_axis_name="c", subcore_axis_name="s", num_cores=NC, num_subcores=NS
    )


def _sc_degree(R, n_groups_tile, G):
    rows_per_sub = R // NS

    @functools.partial(
        pl.kernel,
        out_type=jax.ShapeDtypeStruct((NC * R,), jnp.float32),
        mesh=_mesh(),
        compiler_params=pltpu.CompilerParams(use_tc_tiling_on_sc=False),
        scratch_types=[
            pltpu.VMEM((G, 128), jnp.int32),
            pltpu.VMEM((128,), jnp.float32),
            pltpu.VMEM_SHARED((R,), jnp.float32),
        ],
    )
    def k(dst_hbm, zeros_hbm, out_hbm, dst_v, ones_v, acc):
        c = lax.axis_index("c")
        s = lax.axis_index("s")
        wid = c * NS + s
        for i in range(128 // 16):
            ones_v[pl.ds(i * 16, 16)] = jnp.ones((16,), jnp.float32)
        pltpu.sync_copy(
            zeros_hbm.at[pl.ds(s * rows_per_sub, rows_per_sub)],
            acc.at[pl.ds(s * rows_per_sub, rows_per_sub)],
        )
        plsc.subcore_barrier()

        def body(j, carry):
            gidx = wid * n_groups_tile + j
            pltpu.sync_copy(dst_hbm.at[gidx], dst_v)
            for r in range(G):
                pltpu.sync_copy(ones_v, acc.at[dst_v.at[r]], add=True)
            return carry

        lax.fori_loop(0, n_groups_tile, body, 0)
        plsc.subcore_barrier()
        pltpu.sync_copy(
            acc.at[pl.ds(s * rows_per_sub, rows_per_sub)],
            out_hbm.at[pl.ds(c * R + s * rows_per_sub, rows_per_sub)],
        )

    return k


def _sc_scatter(F, R, n_groups_tile, G):
    rows_per_sub = R // NS

    @functools.partial(
        pl.kernel,
        out_type=jax.ShapeDtypeStruct((NC, R, F), jnp.float32),
        mesh=_mesh(),
        compiler_params=pltpu.CompilerParams(use_tc_tiling_on_sc=False),
        scratch_types=[
            pltpu.VMEM((G, 128), jnp.int32),
            pltpu.VMEM((G, 128), jnp.int32),
            pltpu.VMEM((G, 128, F), jnp.float32),
            pltpu.VMEM_SHARED((R, F), jnp.float32),
            pltpu.SemaphoreType.DMA,
        ],
    )
    def k(src_hbm, dst_hbm, table_hbm, zeros_hbm, out_hbm, src_v, dst_v, rows_v, acc, gsem):
        c = lax.axis_index("c")
        s = lax.axis_index("s")
        wid = c * NS + s
        pltpu.sync_copy(
            zeros_hbm.at[pl.ds(s * rows_per_sub, rows_per_sub)],
            acc.at[pl.ds(s * rows_per_sub, rows_per_sub)],
        )
        plsc.subcore_barrier()

        def body(j, carry):
            gidx = wid * n_groups_tile + j
            pltpu.sync_copy(src_hbm.at[gidx], src_v)
            pltpu.sync_copy(dst_hbm.at[gidx], dst_v)
            descs = [
                pltpu.async_copy(table_hbm.at[src_v.at[r]], rows_v.at[r], gsem)
                for r in range(G)
            ]
            for d in descs:
                d.wait()
            for r in range(G):
                pltpu.sync_copy(rows_v.at[r], acc.at[dst_v.at[r]], add=True)
            return carry

        lax.fori_loop(0, n_groups_tile, body, 0)
        plsc.subcore_barrier()
        pltpu.sync_copy(
            acc.at[pl.ds(s * rows_per_sub, rows_per_sub)],
            out_hbm.at[c, pl.ds(s * rows_per_sub, rows_per_sub)],
        )

    return k


def _tc1_body(dega_ref, degb_ref, x_ref, w_ref, dinv_ref, g_ref):
    deg = dega_ref[...] + degb_ref[...] + 1.0
    dinv = lax.rsqrt(deg)
    dinv_ref[...] = dinv
    h = jnp.dot(x_ref[...], w_ref[...], preferred_element_type=jnp.float32,
                precision=lax.Precision.HIGHEST)
    g_ref[...] = h * dinv


def _tc2_body(a0_ref, a1_ref, g1_ref, dinv_ref, b1_ref, u_ref):
    dinv = dinv_ref[...]
    s = a0_ref[...] + a1_ref[...] + g1_ref[...]
    o1 = jnp.maximum(s * dinv + b1_ref[...], 0.0)
    u_ref[...] = o1 * dinv


def _tc3_body(c0_ref, c1_ref, u_ref, dinv_ref, b2_ref, w2_ref, out_ref):
    su = c0_ref[...] + c1_ref[...] + u_ref[...]
    h2 = jnp.dot(su, w2_ref[...], preferred_element_type=jnp.float32,
                 precision=lax.Precision.HIGHEST)
    pre = h2 * dinv_ref[...] + b2_ref[...]
    m = jnp.max(pre, axis=1, keepdims=True)
    z = pre - m
    lse = jnp.log(jnp.sum(jnp.exp(z), axis=1, keepdims=True))
    out_ref[...] = z - lse


def kernel(x, edge_index, W1, b1, W2, b2):
    N, D = x.shape
    F1 = W1.shape[1]
    F2 = W2.shape[1]
    E = edge_index.shape[1]

    G1, G2 = 4, 16
    unit = NW * 128 * G2
    EP = ((E + unit - 1) // unit) * unit
    R = ((N + NS * 128) // (NS * 128)) * (NS * 128)
    rows_tile = EP // 128 // NW

    ei = edge_index.astype(jnp.int32)
    pad = EP - E
    src = jnp.concatenate([ei[0], jnp.zeros((pad,), jnp.int32)])
    dst = jnp.concatenate([ei[1], jnp.full((pad,), N, jnp.int32)])
    src1 = src.reshape(EP // 128 // G1, G1, 128)
    dst1 = dst.reshape(EP // 128 // G1, G1, 128)
    src2 = src.reshape(EP // 128 // G2, G2, 128)
    dst2 = dst.reshape(EP // 128 // G2, G2, 128)

    zeros1 = jnp.zeros((R,), jnp.float32)
    zerosF1 = jnp.zeros((R, F1), jnp.float32)

    deg_parts = _sc_degree(R, rows_tile // G2, G2)(dst2, zeros1).reshape(NC, R)

    B = 2000
    grid = (N // B,)
    dinv, g1 = pl.pallas_call(
        _tc1_body,
        grid=grid,
        in_specs=[
            pl.BlockSpec((B, 1), lambda i: (i, 0)),
            pl.BlockSpec((B, 1), lambda i: (i, 0)),
            pl.BlockSpec((B, D), lambda i: (i, 0)),
            pl.BlockSpec((D, F1), lambda i: (0, 0)),
        ],
        out_specs=[
            pl.BlockSpec((B, 1), lambda i: (i, 0)),
            pl.BlockSpec((B, F1), lambda i: (i, 0)),
        ],
        out_shape=[
            jax.ShapeDtypeStruct((N, 1), jnp.float32),
            jax.ShapeDtypeStruct((N, F1), jnp.float32),
        ],
    )(deg_parts[0, :N, None], deg_parts[1, :N, None], x, W1)

    s1 = _sc_scatter(F1, R, rows_tile // G1, G1)(src1, dst1, g1, zerosF1)

    u = pl.pallas_call(
        _tc2_body,
        grid=grid,
        in_specs=[
            pl.BlockSpec((B, F1), lambda i: (i, 0)),
            pl.BlockSpec((B, F1), lambda i: (i, 0)),
            pl.BlockSpec((B, F1), lambda i: (i, 0)),
            pl.BlockSpec((B, 1), lambda i: (i, 0)),
            pl.BlockSpec((1, F1), lambda i: (0, 0)),
        ],
        out_specs=pl.BlockSpec((B, F1), lambda i: (i, 0)),
        out_shape=jax.ShapeDtypeStruct((N, F1), jnp.float32),
    )(s1[0, :N], s1[1, :N], g1, dinv, b1[None, :])

    s2 = _sc_scatter(F1, R, rows_tile // G1, G1)(src1, dst1, u, zerosF1)

    out = pl.pallas_call(
        _tc3_body,
        grid=grid,
        in_specs=[
            pl.BlockSpec((B, F1), lambda i: (i, 0)),
            pl.BlockSpec((B, F1), lambda i: (i, 0)),
            pl.BlockSpec((B, F1), lambda i: (i, 0)),
            pl.BlockSpec((B, 1), lambda i: (i, 0)),
            pl.BlockSpec((1, F2), lambda i: (0, 0)),
            pl.BlockSpec((F1, F2), lambda i: (0, 0)),
        ],
        out_specs=pl.BlockSpec((B, F2), lambda i: (i, 0)),
        out_shape=jax.ShapeDtypeStruct((N, F2), jnp.float32),
    )(s2[0, :N], s2[1, :N], u, dinv, b2[None, :], W2)

    return out


# speedup vs baseline: 44.3905x; 1.0881x over previous
import functools

import jax
import jax.numpy as jnp
from jax import lax
from jax.experimental import pallas as pl
from jax.experimental.pallas import tpu as pltpu
from jax.experimental.pallas import tpu_sc as plsc

NC = 2
NS = 16
NW = NC * NS
ZROWS = 784


def _mesh():
    return plsc.VectorSubcoreMesh(
        core_axis_name="c", subcore_axis_name="s", num_cores=NC, num_subcores=NS
    )


def _tile_share(wid, n_groups):
    base = n_groups // NW
    rem = n_groups % NW
    count = base + jnp.where(wid < rem, 1, 0)
    start = wid * base + jnp.minimum(wid, rem)
    return start, count


def _sc_degree(R, G, n_groups):
    rows_per_sub = R // NS

    @functools.partial(
        pl.kernel,
        out_type=jax.ShapeDtypeStruct((NC * R,), jnp.float32),
        mesh=_mesh(),
        compiler_params=pltpu.CompilerParams(use_tc_tiling_on_sc=False),
        scratch_types=[
            pltpu.VMEM((G, 128), jnp.int32),
            pltpu.VMEM((128,), jnp.float32),
            pltpu.VMEM((ZROWS,), jnp.float32),
            pltpu.VMEM_SHARED((R,), jnp.float32),
        ],
    )
    def k(dst_hbm, out_hbm, dst_v, ones_v, zbuf, acc):
        c = lax.axis_index("c")
        s = lax.axis_index("s")
        wid = c * NS + s
        for i in range(128 // 16):
            ones_v[pl.ds(i * 16, 16)] = jnp.ones((16,), jnp.float32)

        def zfill(i, carry):
            zbuf[pl.ds(i * 16, 16)] = jnp.zeros((16,), jnp.float32)
            return carry

        lax.fori_loop(0, ZROWS // 16, zfill, 0)
        for blk in range(rows_per_sub // ZROWS):
            pltpu.sync_copy(zbuf, acc.at[pl.ds(s * rows_per_sub + blk * ZROWS, ZROWS)])
        plsc.subcore_barrier()
        start, count = _tile_share(wid, n_groups)

        def body(j, carry):
            pltpu.sync_copy(dst_hbm.at[start + j], dst_v)
            for r in range(G):
                pltpu.sync_copy(ones_v, acc.at[dst_v.at[r]], add=True)
            return carry

        lax.fori_loop(0, count, body, 0)
        plsc.subcore_barrier()
        pltpu.sync_copy(
            acc.at[pl.ds(s * rows_per_sub, rows_per_sub)],
            out_hbm.at[pl.ds(c * R + s * rows_per_sub, rows_per_sub)],
        )

    return k


def _sc_scatter(F, R, G, n_groups):
    rows_per_sub = R // NS

    @functools.partial(
        pl.kernel,
        out_type=jax.ShapeDtypeStruct((NC, R, F), jnp.float32),
        mesh=_mesh(),
        compiler_params=pltpu.CompilerParams(use_tc_tiling_on_sc=False),
        scratch_types=[
            pltpu.VMEM((G, 128), jnp.int32),
            pltpu.VMEM((G, 128), jnp.int32),
            pltpu.VMEM((G, 128, F), jnp.float32),
            pltpu.VMEM((ZROWS, F), jnp.float32),
            pltpu.VMEM_SHARED((R, F), jnp.float32),
            pltpu.SemaphoreType.DMA,
        ],
    )
    def k(src_hbm, dst_hbm, table_hbm, out_hbm, src_v, dst_v, rows_v, zbuf, acc, gsem):
        c = lax.axis_index("c")
        s = lax.axis_index("s")
        wid = c * NS + s

        def zfill(i, carry):
            zbuf[i, :] = jnp.zeros((16,), jnp.float32)
            return carry

        lax.fori_loop(0, ZROWS, zfill, 0)
        for blk in range(rows_per_sub // ZROWS):
            pltpu.sync_copy(
                zbuf,
                acc.at[pl.ds(s * rows_per_sub + blk * ZROWS, ZROWS)],
            )
        plsc.subcore_barrier()
        start, count = _tile_share(wid, n_groups)

        def body(j, carry):
            gidx = start + j
            pltpu.sync_copy(src_hbm.at[gidx], src_v)
            pltpu.sync_copy(dst_hbm.at[gidx], dst_v)
            descs = [
                pltpu.async_copy(table_hbm.at[src_v.at[r]], rows_v.at[r], gsem)
                for r in range(G)
            ]
            for d in descs:
                d.wait()
            for r in range(G):
                pltpu.sync_copy(rows_v.at[r], acc.at[dst_v.at[r]], add=True)
            return carry

        lax.fori_loop(0, count, body, 0)
        plsc.subcore_barrier()
        pltpu.sync_copy(
            acc.at[pl.ds(s * rows_per_sub, rows_per_sub)],
            out_hbm.at[c, pl.ds(s * rows_per_sub, rows_per_sub)],
        )

    return k


def _tc1_body(dega_ref, degb_ref, x_ref, w_ref, dinv_ref, g_ref):
    deg = dega_ref[0] + degb_ref[0] + 1.0
    dinv = lax.rsqrt(deg)
    dinv_ref[...] = dinv
    h = jnp.dot(x_ref[...], w_ref[...], preferred_element_type=jnp.float32,
                precision=lax.Precision.HIGHEST)
    g_ref[...] = h * dinv


def _tc2_body(s1_ref, s1b_ref, g1_ref, dinv_ref, b1_ref, u_ref):
    dinv = dinv_ref[...]
    s = s1_ref[0] + s1b_ref[0] + g1_ref[...]
    o1 = jnp.maximum(s * dinv + b1_ref[...], 0.0)
    u_ref[...] = o1 * dinv


def _tc3_body(c0_ref, c1_ref, u_ref, dinv_ref, b2_ref, w2_ref, out_ref):
    su = c0_ref[0] + c1_ref[0] + u_ref[...]
    h2 = jnp.dot(su, w2_ref[...], preferred_element_type=jnp.float32,
                 precision=lax.Precision.HIGHEST)
    pre = h2 * dinv_ref[...] + b2_ref[...]
    m = jnp.max(pre, axis=1, keepdims=True)
    z = pre - m
    lse = jnp.log(jnp.sum(jnp.exp(z), axis=1, keepdims=True))
    out_ref[...] = z - lse


def kernel(x, edge_index, W1, b1, W2, b2):
    N, D = x.shape
    F1 = W1.shape[1]
    F2 = W2.shape[1]
    E = edge_index.shape[1]
    assert E % 128 == 0
    n_rows = E // 128

    GS = 4
    GD = 25
    assert n_rows % GS == 0 and n_rows % GD == 0
    R = ((N + NS * ZROWS - 1) // (NS * ZROWS)) * (NS * ZROWS)

    ei = edge_index.astype(jnp.int32)
    src_s = ei[0].reshape(n_rows // GS, GS, 128)
    dst_s = ei[1].reshape(n_rows // GS, GS, 128)
    dst_d = ei[1].reshape(n_rows // GD, GD, 128)

    deg_parts = _sc_degree(R, GD, n_rows // GD)(dst_d).reshape(NC, R, 1)

    B = 2000
    grid = (N // B,)
    dinv, g1 = pl.pallas_call(
        _tc1_body,
        grid=grid,
        in_specs=[
            pl.BlockSpec((1, B, 1), lambda i: (0, i, 0)),
            pl.BlockSpec((1, B, 1), lambda i: (1, i, 0)),
            pl.BlockSpec((B, D), lambda i: (i, 0)),
            pl.BlockSpec((D, F1), lambda i: (0, 0)),
        ],
        out_specs=[
            pl.BlockSpec((B, 1), lambda i: (i, 0)),
            pl.BlockSpec((B, F1), lambda i: (i, 0)),
        ],
        out_shape=[
            jax.ShapeDtypeStruct((N, 1), jnp.float32),
            jax.ShapeDtypeStruct((N, F1), jnp.float32),
        ],
    )(deg_parts, deg_parts, x, W1)

    s1 = _sc_scatter(F1, R, GS, n_rows // GS)(src_s, dst_s, g1)

    u = pl.pallas_call(
        _tc2_body,
        grid=grid,
        in_specs=[
            pl.BlockSpec((1, B, F1), lambda i: (0, i, 0)),
            pl.BlockSpec((1, B, F1), lambda i: (1, i, 0)),
            pl.BlockSpec((B, F1), lambda i: (i, 0)),
            pl.BlockSpec((B, 1), lambda i: (i, 0)),
            pl.BlockSpec((1, F1), lambda i: (0, 0)),
        ],
        out_specs=pl.BlockSpec((B, F1), lambda i: (i, 0)),
        out_shape=jax.ShapeDtypeStruct((N, F1), jnp.float32),
    )(s1, s1, g1, dinv, b1[None, :])

    s2 = _sc_scatter(F1, R, GS, n_rows // GS)(src_s, dst_s, u)

    out = pl.pallas_call(
        _tc3_body,
        grid=grid,
        in_specs=[
            pl.BlockSpec((1, B, F1), lambda i: (0, i, 0)),
            pl.BlockSpec((1, B, F1), lambda i: (1, i, 0)),
            pl.BlockSpec((B, F1), lambda i: (i, 0)),
            pl.BlockSpec((B, 1), lambda i: (i, 0)),
            pl.BlockSpec((1, F2), lambda i: (0, 0)),
            pl.BlockSpec((F1, F2), lambda i: (0, 0)),
        ],
        out_specs=pl.BlockSpec((B, F2), lambda i: (i, 0)),
        out_shape=jax.ShapeDtypeStruct((N, F2), jnp.float32),
    )(s2, s2, u, dinv, b2[None, :], W2)

    return out


# speedup vs baseline: 57.3794x; 1.2926x over previous
import functools

import jax
import jax.numpy as jnp
from jax import lax
from jax.experimental import pallas as pl
from jax.experimental.pallas import tpu as pltpu
from jax.experimental.pallas import tpu_sc as plsc

NC = 2
NS = 16
NW = NC * NS


def _mesh():
    return plsc.VectorSubcoreMesh(
        core_axis_name="c", subcore_axis_name="s", num_cores=NC, num_subcores=NS
    )


def _tile_share(wid, n_groups):
    base = n_groups // NW
    rem = n_groups % NW
    count = base + jnp.where(wid < rem, 1, 0)
    start = wid * base + jnp.minimum(wid, rem)
    return start, count


def _sc_degree(R, G, n_groups, zrows):
    rows_per_sub = R // NS

    @functools.partial(
        pl.kernel,
        out_type=jax.ShapeDtypeStruct((NC * R,), jnp.float32),
        mesh=_mesh(),
        compiler_params=pltpu.CompilerParams(use_tc_tiling_on_sc=False),
        scratch_types=[
            pltpu.VMEM((G, 128), jnp.int32),
            pltpu.VMEM((128,), jnp.float32),
            pltpu.VMEM((zrows,), jnp.float32),
            pltpu.VMEM_SHARED((R,), jnp.float32),
        ],
    )
    def k(dst_hbm, out_hbm, dst_v, ones_v, zbuf, acc):
        c = lax.axis_index("c")
        s = lax.axis_index("s")
        wid = c * NS + s
        for i in range(128 // 16):
            ones_v[pl.ds(i * 16, 16)] = jnp.ones((16,), jnp.float32)

        def zfill(i, carry):
            zbuf[pl.ds(i * 16, 16)] = jnp.zeros((16,), jnp.float32)
            return carry

        lax.fori_loop(0, zrows // 16, zfill, 0)
        for blk in range(rows_per_sub // zrows):
            pltpu.sync_copy(zbuf, acc.at[pl.ds(s * rows_per_sub + blk * zrows, zrows)])
        plsc.subcore_barrier()
        start, count = _tile_share(wid, n_groups)

        def body(j, carry):
            pltpu.sync_copy(dst_hbm.at[start + j], dst_v)
            for r in range(G):
                pltpu.sync_copy(ones_v, acc.at[dst_v.at[r]], add=True)
            return carry

        lax.fori_loop(0, count, body, 0)
        plsc.subcore_barrier()
        pltpu.sync_copy(
            acc.at[pl.ds(s * rows_per_sub, rows_per_sub)],
            out_hbm.at[pl.ds(c * R + s * rows_per_sub, rows_per_sub)],
        )

    return k


def _sc_scatter(F, R, G, n_groups, zrows):
    rows_per_sub = R // NS
    n_pairs = n_groups // 2
    assert n_groups % 2 == 0 and n_pairs >= NW

    @functools.partial(
        pl.kernel,
        out_type=jax.ShapeDtypeStruct((NC, R, F), jnp.float32),
        mesh=_mesh(),
        compiler_params=pltpu.CompilerParams(use_tc_tiling_on_sc=False),
        scratch_types=[
            pltpu.VMEM((G, 128), jnp.int32),
            pltpu.VMEM((G, 128), jnp.int32),
            pltpu.VMEM((G, 128), jnp.int32),
            pltpu.VMEM((G, 128), jnp.int32),
            pltpu.VMEM((G, 128, F), jnp.float32),
            pltpu.VMEM((G, 128, F), jnp.float32),
            pltpu.VMEM((zrows, F), jnp.float32),
            pltpu.VMEM_SHARED((R, F), jnp.float32),
            pltpu.SemaphoreType.DMA,
            pltpu.SemaphoreType.DMA,
        ],
    )
    def k(src_hbm, dst_hbm, table_hbm, out_hbm,
          src0, dst0, src1, dst1, rows0, rows1, zbuf, acc, gsem0, gsem1):
        c = lax.axis_index("c")
        s = lax.axis_index("s")
        wid = c * NS + s

        def zfill(i, carry):
            zbuf[i, :] = jnp.zeros((16,), jnp.float32)
            return carry

        lax.fori_loop(0, zrows, zfill, 0)
        for blk in range(rows_per_sub // zrows):
            pltpu.sync_copy(
                zbuf,
                acc.at[pl.ds(s * rows_per_sub + blk * zrows, zrows)],
            )
        plsc.subcore_barrier()
        pstart, pcount = _tile_share(wid, n_pairs)
        g_base = pstart * 2
        g_last = g_base + pcount * 2 - 1

        def fire(src_v, rows_v, sem):
            return [
                pltpu.async_copy(table_hbm.at[src_v.at[r]], rows_v.at[r], sem)
                for r in range(G)
            ]

        def scat(rows_v, dst_v):
            for r in range(G):
                pltpu.sync_copy(rows_v.at[r], acc.at[dst_v.at[r]], add=True)

        def pair(kk, carry):
            g = g_base + 2 * kk
            pltpu.sync_copy(src_hbm.at[g], src0)
            pltpu.sync_copy(dst_hbm.at[g], dst0)
            d0 = fire(src0, rows0, gsem0)
            pltpu.sync_copy(src_hbm.at[g + 1], src1)
            pltpu.sync_copy(dst_hbm.at[g + 1], dst1)
            d1 = fire(src1, rows1, gsem1)
            for d in d0:
                d.wait()
            scat(rows0, dst0)
            for d in d1:
                d.wait()
            scat(rows1, dst1)
            return carry

        lax.fori_loop(0, pcount, pair, 0)
        plsc.subcore_barrier()
        pltpu.sync_copy(
            acc.at[pl.ds(s * rows_per_sub, rows_per_sub)],
            out_hbm.at[c, pl.ds(s * rows_per_sub, rows_per_sub)],
        )

    return k


def _tc1_body(dega_ref, degb_ref, x_ref, w_ref, dinv_ref, g_ref):
    deg = dega_ref[0] + degb_ref[0] + 1.0
    dinv = lax.rsqrt(deg)
    dinv_ref[...] = dinv
    h = jnp.dot(x_ref[...], w_ref[...], preferred_element_type=jnp.float32,
                precision=lax.Precision.HIGHEST)
    g_ref[...] = h * dinv


def _tc2_body(s1_ref, s1b_ref, g1_ref, dinv_ref, b1_ref, u_ref):
    dinv = dinv_ref[...]
    s = s1_ref[0] + s1b_ref[0] + g1_ref[...]
    o1 = jnp.maximum(s * dinv + b1_ref[...], 0.0)
    u_ref[...] = o1 * dinv


def _tc3_body(c0_ref, c1_ref, u_ref, dinv_ref, b2_ref, w2_ref, out_ref):
    su = c0_ref[0] + c1_ref[0] + u_ref[...]
    h2 = jnp.dot(su, w2_ref[...], preferred_element_type=jnp.float32,
                 precision=lax.Precision.HIGHEST)
    pre = h2 * dinv_ref[...] + b2_ref[...]
    m = jnp.max(pre, axis=1, keepdims=True)
    z = pre - m
    lse = jnp.log(jnp.sum(jnp.exp(z), axis=1, keepdims=True))
    out_ref[...] = z - lse


def kernel(x, edge_index, W1, b1, W2, b2):
    N, D = x.shape
    F1 = W1.shape[1]
    F2 = W2.shape[1]
    E = edge_index.shape[1]
    assert E % 128 == 0
    n_rows = E // 128

    GS = 5
    GD = 25
    assert n_rows % GS == 0 and n_rows % GD == 0
    Rd = ((N + NS * 128 - 1) // (NS * 128)) * (NS * 128)
    assert N % NS == 0

    def _zrows(rps, cap=1024):
        z = 1
        for d in range(1, cap + 1):
            if rps % d == 0:
                z = d
        return z

    ei = edge_index.astype(jnp.int32)
    src_s = ei[0].reshape(n_rows // GS, GS, 128)
    dst_s = ei[1].reshape(n_rows // GS, GS, 128)
    dst_d = ei[1].reshape(n_rows // GD, GD, 128)

    deg_parts = _sc_degree(Rd, GD, n_rows // GD, _zrows(Rd // NS))(dst_d)
    deg_parts = deg_parts.reshape(NC, Rd, 1)

    B = 2000
    grid = (N // B,)
    dinv, g1 = pl.pallas_call(
        _tc1_body,
        grid=grid,
        in_specs=[
            pl.BlockSpec((1, B, 1), lambda i: (0, i, 0)),
            pl.BlockSpec((1, B, 1), lambda i: (1, i, 0)),
            pl.BlockSpec((B, D), lambda i: (i, 0)),
            pl.BlockSpec((D, F1), lambda i: (0, 0)),
        ],
        out_specs=[
            pl.BlockSpec((B, 1), lambda i: (i, 0)),
            pl.BlockSpec((B, F1), lambda i: (i, 0)),
        ],
        out_shape=[
            jax.ShapeDtypeStruct((N, 1), jnp.float32),
            jax.ShapeDtypeStruct((N, F1), jnp.float32),
        ],
    )(deg_parts, deg_parts, x, W1)

    zs = _zrows(N // NS, cap=125)
    scat = _sc_scatter(F1, N, GS, n_rows // GS, zs)
    s1 = scat(src_s, dst_s, g1)

    u = pl.pallas_call(
        _tc2_body,
        grid=grid,
        in_specs=[
            pl.BlockSpec((1, B, F1), lambda i: (0, i, 0)),
            pl.BlockSpec((1, B, F1), lambda i: (1, i, 0)),
            pl.BlockSpec((B, F1), lambda i: (i, 0)),
            pl.BlockSpec((B, 1), lambda i: (i, 0)),
            pl.BlockSpec((1, F1), lambda i: (0, 0)),
        ],
        out_specs=pl.BlockSpec((B, F1), lambda i: (i, 0)),
        out_shape=jax.ShapeDtypeStruct((N, F1), jnp.float32),
    )(s1, s1, g1, dinv, b1[None, :])

    s2 = scat(src_s, dst_s, u)

    out = pl.pallas_call(
        _tc3_body,
        grid=grid,
        in_specs=[
            pl.BlockSpec((1, B, F1), lambda i: (0, i, 0)),
            pl.BlockSpec((1, B, F1), lambda i: (1, i, 0)),
            pl.BlockSpec((B, F1), lambda i: (i, 0)),
            pl.BlockSpec((B, 1), lambda i: (i, 0)),
            pl.BlockSpec((1, F2), lambda i: (0, 0)),
            pl.BlockSpec((F1, F2), lambda i: (0, 0)),
        ],
        out_specs=pl.BlockSpec((B, F2), lambda i: (i, 0)),
        out_shape=jax.ShapeDtypeStruct((N, F2), jnp.float32),
    )(s2, s2, u, dinv, b2[None, :], W2)

    return out


# speedup vs baseline: 62.2674x; 1.0852x over previous
import functools

import jax
import jax.numpy as jnp
from jax import lax
from jax.experimental import pallas as pl
from jax.experimental.pallas import tpu as pltpu
from jax.experimental.pallas import tpu_sc as plsc

NC = 2
NS = 16
NW = NC * NS


def _mesh():
    return plsc.VectorSubcoreMesh(
        core_axis_name="c", subcore_axis_name="s", num_cores=NC, num_subcores=NS
    )


def _tile_share(wid, n_groups):
    base = n_groups // NW
    rem = n_groups % NW
    count = base + jnp.where(wid < rem, 1, 0)
    start = wid * base + jnp.minimum(wid, rem)
    return start, count


def _sc_degree(R, G, n_groups, zrows):
    rows_per_sub = R // NS

    @functools.partial(
        pl.kernel,
        out_type=jax.ShapeDtypeStruct((NC * R,), jnp.float32),
        mesh=_mesh(),
        compiler_params=pltpu.CompilerParams(use_tc_tiling_on_sc=False),
        scratch_types=[
            pltpu.VMEM((G, 128), jnp.int32),
            pltpu.VMEM((128,), jnp.float32),
            pltpu.VMEM((zrows,), jnp.float32),
            pltpu.VMEM_SHARED((R,), jnp.float32),
        ],
    )
    def k(dst_hbm, out_hbm, dst_v, ones_v, zbuf, acc):
        c = lax.axis_index("c")
        s = lax.axis_index("s")
        wid = c * NS + s
        for i in range(128 // 16):
            ones_v[pl.ds(i * 16, 16)] = jnp.ones((16,), jnp.float32)

        def zfill(i, carry):
            zbuf[pl.ds(i * 16, 16)] = jnp.zeros((16,), jnp.float32)
            return carry

        lax.fori_loop(0, zrows // 16, zfill, 0)
        for blk in range(rows_per_sub // zrows):
            pltpu.sync_copy(zbuf, acc.at[pl.ds(s * rows_per_sub + blk * zrows, zrows)])
        plsc.subcore_barrier()
        start, count = _tile_share(wid, n_groups)

        def body(j, carry):
            pltpu.sync_copy(dst_hbm.at[start + j], dst_v)
            for r in range(G):
                pltpu.sync_copy(ones_v, acc.at[dst_v.at[r]], add=True)
            return carry

        lax.fori_loop(0, count, body, 0)
        plsc.subcore_barrier()
        pltpu.sync_copy(
            acc.at[pl.ds(s * rows_per_sub, rows_per_sub)],
            out_hbm.at[pl.ds(c * R + s * rows_per_sub, rows_per_sub)],
        )

    return k


def _sc_scatter(F, R, G, n_groups, zrows):
    rows_per_sub = R // NS
    n_pairs = n_groups // 2
    assert n_groups % 2 == 0 and n_pairs >= NW

    @functools.partial(
        pl.kernel,
        out_type=jax.ShapeDtypeStruct((NC, R, F), jnp.float32),
        mesh=_mesh(),
        compiler_params=pltpu.CompilerParams(use_tc_tiling_on_sc=False),
        scratch_types=[
            pltpu.VMEM((G, 128), jnp.int32),
            pltpu.VMEM((G, 128), jnp.int32),
            pltpu.VMEM((G, 128), jnp.int32),
            pltpu.VMEM((G, 128), jnp.int32),
            pltpu.VMEM((G, 128, F), jnp.float32),
            pltpu.VMEM((G, 128, F), jnp.float32),
            pltpu.VMEM((zrows, F), jnp.float32),
            pltpu.VMEM_SHARED((R, F), jnp.float32),
            pltpu.SemaphoreType.DMA,
            pltpu.SemaphoreType.DMA,
        ],
    )
    def k(src_hbm, dst_hbm, table_hbm, out_hbm,
          src0, dst0, src1, dst1, rows0, rows1, zbuf, acc, gsem0, gsem1):
        c = lax.axis_index("c")
        s = lax.axis_index("s")
        wid = c * NS + s

        def zfill(i, carry):
            zbuf[i, :] = jnp.zeros((16,), jnp.float32)
            return carry

        lax.fori_loop(0, zrows, zfill, 0)
        for blk in range(rows_per_sub // zrows):
            pltpu.sync_copy(
                zbuf,
                acc.at[pl.ds(s * rows_per_sub + blk * zrows, zrows)],
            )
        plsc.subcore_barrier()
        pstart, pcount = _tile_share(wid, n_pairs)
        g_base = pstart * 2
        g_last = g_base + pcount * 2 - 1

        def fire(src_v, rows_v, sem):
            return [
                pltpu.async_copy(table_hbm.at[src_v.at[r]], rows_v.at[r], sem)
                for r in range(G)
            ]

        def scat(rows_v, dst_v):
            for r in range(G):
                pltpu.sync_copy(rows_v.at[r], acc.at[dst_v.at[r]], add=True)

        def pair(kk, carry):
            g = g_base + 2 * kk
            pltpu.sync_copy(src_hbm.at[g], src0)
            pltpu.sync_copy(dst_hbm.at[g], dst0)
            d0 = fire(src0, rows0, gsem0)
            pltpu.sync_copy(src_hbm.at[g + 1], src1)
            pltpu.sync_copy(dst_hbm.at[g + 1], dst1)
            d1 = fire(src1, rows1, gsem1)
            for d in d0:
                d.wait()
            scat(rows0, dst0)
            for d in d1:
                d.wait()
            scat(rows1, dst1)
            return carry

        lax.fori_loop(0, pcount, pair, 0)
        plsc.subcore_barrier()
        pltpu.sync_copy(
            acc.at[pl.ds(s * rows_per_sub, rows_per_sub)],
            out_hbm.at[c, pl.ds(s * rows_per_sub, rows_per_sub)],
        )

    return k


def _tc1_body(dega_ref, degb_ref, x_ref, w_ref, dinv_ref, g_ref):
    deg = dega_ref[...] + degb_ref[...] + 1.0
    dinv = lax.rsqrt(deg)[:, None]
    dinv_ref[...] = dinv
    h = jnp.dot(x_ref[...], w_ref[...], preferred_element_type=jnp.float32,
                precision=lax.Precision.HIGHEST)
    g_ref[...] = h * dinv


def _tc2_body(s1_ref, s1b_ref, g1_ref, dinv_ref, b1_ref, u_ref):
    dinv = dinv_ref[...]
    s = s1_ref[0] + s1b_ref[0] + g1_ref[...]
    o1 = jnp.maximum(s * dinv + b1_ref[...], 0.0)
    u_ref[...] = o1 * dinv


def _tc3_body(c0_ref, c1_ref, u_ref, dinv_ref, b2_ref, w2_ref, out_ref):
    su = c0_ref[0] + c1_ref[0] + u_ref[...]
    h2 = jnp.dot(su, w2_ref[...], preferred_element_type=jnp.float32,
                 precision=lax.Precision.HIGHEST)
    pre = h2 * dinv_ref[...] + b2_ref[...]
    m = jnp.max(pre, axis=1, keepdims=True)
    z = pre - m
    lse = jnp.log(jnp.sum(jnp.exp(z), axis=1, keepdims=True))
    out_ref[...] = z - lse


def kernel(x, edge_index, W1, b1, W2, b2):
    N, D = x.shape
    F1 = W1.shape[1]
    F2 = W2.shape[1]
    E = edge_index.shape[1]
    assert E % 128 == 0
    n_rows = E // 128

    GS = 5
    GD = 25
    assert n_rows % GS == 0 and n_rows % GD == 0
    Rd = ((N + NS * 128 - 1) // (NS * 128)) * (NS * 128)
    assert N % NS == 0

    def _zrows(rps, cap=1024):
        z = 1
        for d in range(1, cap + 1):
            if rps % d == 0:
                z = d
        return z

    ei = edge_index.astype(jnp.int32)
    src_s = ei[0].reshape(n_rows // GS, GS, 128)
    dst_s = ei[1].reshape(n_rows // GS, GS, 128)
    dst_d = ei[1].reshape(n_rows // GD, GD, 128)

    deg_flat = _sc_degree(Rd, GD, n_rows // GD, _zrows(Rd // NS))(dst_d)

    B1 = 2048
    nb1 = Rd // B1
    dinv, g1 = pl.pallas_call(
        _tc1_body,
        grid=(nb1,),
        in_specs=[
            pl.BlockSpec((B1,), lambda i: (i,)),
            pl.BlockSpec((B1,), lambda i: (nb1 + i,)),
            pl.BlockSpec((B1, D), lambda i: (i, 0)),
            pl.BlockSpec((D, F1), lambda i: (0, 0)),
        ],
        out_specs=[
            pl.BlockSpec((B1, 1), lambda i: (i, 0)),
            pl.BlockSpec((B1, F1), lambda i: (i, 0)),
        ],
        out_shape=[
            jax.ShapeDtypeStruct((N, 1), jnp.float32),
            jax.ShapeDtypeStruct((N, F1), jnp.float32),
        ],
    )(deg_flat, deg_flat, x, W1)

    B = 2000
    grid = (N // B,)

    zs = _zrows(N // NS, cap=125)
    scat = _sc_scatter(F1, N, GS, n_rows // GS, zs)
    s1 = scat(src_s, dst_s, g1)

    u = pl.pallas_call(
        _tc2_body,
        grid=grid,
        in_specs=[
            pl.BlockSpec((1, B, F1), lambda i: (0, i, 0)),
            pl.BlockSpec((1, B, F1), lambda i: (1, i, 0)),
            pl.BlockSpec((B, F1), lambda i: (i, 0)),
            pl.BlockSpec((B, 1), lambda i: (i, 0)),
            pl.BlockSpec((1, F1), lambda i: (0, 0)),
        ],
        out_specs=pl.BlockSpec((B, F1), lambda i: (i, 0)),
        out_shape=jax.ShapeDtypeStruct((N, F1), jnp.float32),
    )(s1, s1, g1, dinv, b1[None, :])

    s2 = scat(src_s, dst_s, u)

    out = pl.pallas_call(
        _tc3_body,
        grid=grid,
        in_specs=[
            pl.BlockSpec((1, B, F1), lambda i: (0, i, 0)),
            pl.BlockSpec((1, B, F1), lambda i: (1, i, 0)),
            pl.BlockSpec((B, F1), lambda i: (i, 0)),
            pl.BlockSpec((B, 1), lambda i: (i, 0)),
            pl.BlockSpec((1, F2), lambda i: (0, 0)),
            pl.BlockSpec((F1, F2), lambda i: (0, 0)),
        ],
        out_specs=pl.BlockSpec((B, F2), lambda i: (i, 0)),
        out_shape=jax.ShapeDtypeStruct((N, F2), jnp.float32),
    )(s2, s2, u, dinv, b2[None, :], W2)

    return out
